# baseline (speedup 1.0000x reference)
import functools

import jax
import jax.numpy as jnp
from jax import lax
from jax.experimental import pallas as pl
from jax.experimental.pallas import tpu as pltpu
from jax.experimental.pallas import tpu_sc as plsc


def _knn_body(x1_ref, x2t_ref, idx_ref, wts_ref, *, m):
    bi = pl.program_id(0)

    x1 = x1_ref[0]
    x2t = x2t_ref[0]
    dot = lax.dot_general(x1, x2t, (((1,), (0,)), ((), ())),
                          preferred_element_type=jnp.float32)
    x1sq = jnp.sum(x1 * x1, axis=1, keepdims=True)
    x2sq = jnp.sum(x2t * x2t, axis=0, keepdims=True)
    d2 = x1sq + x2sq - 2.0 * dot

    tn = d2.shape[0]
    iota = lax.broadcasted_iota(jnp.int32, (tn, m), 1).astype(jnp.float32)
    fm = jnp.float32(m)
    work = d2
    vals = []
    idxs = []
    for k in range(3):
        mv = jnp.min(work, axis=1, keepdims=True)
        mi = jnp.min(jnp.where(work == mv, iota, fm), axis=1, keepdims=True)
        idxs.append(mi)
        vals.append(mv)
        if k < 2:
            work = jnp.where(iota == mi, jnp.inf, work)

    rs = [1.0 / jnp.clip(v, 0.0, 1e-10) for v in vals]
    norm = rs[0] + rs[1] + rs[2]
    wts_ref[0] = jnp.concatenate(
        [rs[0] / norm, rs[1] / norm, rs[2] / norm], axis=1)
    gbase = (bi * m).astype(jnp.float32)
    idx_ref[0] = jnp.concatenate(
        [(g + gbase).astype(jnp.int32).T for g in idxs], axis=0)


def _make_sc_gather(n_rows, d, n_workers):
    mesh = plsc.VectorSubcoreMesh(core_axis_name="c", subcore_axis_name="s")
    rpw = n_rows // n_workers
    blk = 1024
    grp = 128
    nblk = rpw // blk

    @functools.partial(
        pl.kernel, mesh=mesh,
        out_type=jax.ShapeDtypeStruct((n_rows, d), jnp.float32),
        compiler_params=pltpu.CompilerParams(use_tc_tiling_on_sc=False),
        scratch_types=[
            pltpu.VMEM((blk,), jnp.int32),
            pltpu.VMEM((blk, d), jnp.float32),
            pltpu.SemaphoreType.DMA,
        ],
    )
    def gather(idx_hbm, table_hbm, out_hbm, idx_v, rows_v, sem):
        wid = lax.axis_index("s") * 2 + lax.axis_index("c")
        base = wid * rpw

        def body(i, carry):
            off = base + i * blk
            pltpu.sync_copy(idx_hbm.at[pl.ds(off, blk)], idx_v)
            copies = [
                pltpu.async_copy(table_hbm.at[idx_v.at[pl.ds(j * grp, grp)]],
                                 rows_v.at[pl.ds(j * grp, grp)], sem)
                for j in range(blk // grp)
            ]
            for cp in copies:
                cp.wait()
            pltpu.sync_copy(rows_v, out_hbm.at[pl.ds(off, blk)])
            return carry

        lax.fori_loop(0, nblk, body, 0)

    return gather


def _y0p_body(p1_ref, w0tb_ref, b0_ref, y0p_ref):
    y0p_ref[0] = lax.dot_general(p1_ref[0], w0tb_ref[...],
                                 (((1,), (0,)), ((), ())),
                                 preferred_element_type=jnp.float32) + b0_ref[...]


def _interp_mlp0_body(g_ref, w_ref, y0p_ref, w0ta_ref, y0_ref, s_ref, q_ref):
    bi = pl.program_id(0)
    ti = pl.program_id(1)
    w = w_ref[0]
    interp = (w[:, 0:1] * g_ref[0, 0]
              + w[:, 1:2] * g_ref[0, 1]
              + w[:, 2:3] * g_ref[0, 2])
    y0 = y0p_ref[0] + lax.dot_general(interp, w0ta_ref[...],
                                      (((1,), (0,)), ((), ())),
                                      preferred_element_type=jnp.float32)
    y0_ref[0] = y0

    @pl.when(jnp.logical_and(bi == 0, ti == 0))
    def _init():
        s_ref[...] = jnp.zeros_like(s_ref)
        q_ref[...] = jnp.zeros_like(q_ref)

    s_ref[...] += jnp.sum(y0, axis=0, keepdims=True)
    q_ref[...] += jnp.sum(y0 * y0, axis=0, keepdims=True)


def _mlp1_body(y0_ref, s0_ref, q0_ref, g0_ref, be0_ref, w1t_ref, b1_ref,
               y1_ref, s_ref, q_ref, *, inv_cnt):
    bi = pl.program_id(0)
    ti = pl.program_id(1)
    mean0 = s0_ref[...] * inv_cnt
    var0 = q0_ref[...] * inv_cnt - mean0 * mean0
    a0 = g0_ref[...] / jnp.sqrt(var0 + 1e-5)
    c0 = be0_ref[...] - mean0 * a0
    h = jnp.maximum(y0_ref[0] * a0 + c0, 0.0)
    y1 = lax.dot_general(h, w1t_ref[...], (((1,), (0,)), ((), ())),
                         preferred_element_type=jnp.float32) + b1_ref[...]
    y1_ref[0] = y1

    @pl.when(jnp.logical_and(bi == 0, ti == 0))
    def _init():
        s_ref[...] = jnp.zeros_like(s_ref)
        q_ref[...] = jnp.zeros_like(q_ref)

    s_ref[...] += jnp.sum(y1, axis=0, keepdims=True)
    q_ref[...] += jnp.sum(y1 * y1, axis=0, keepdims=True)


def _finalize_body(y1_ref, s1_ref, q1_ref, g1_ref, be1_ref, out_ref, *,
                   inv_cnt):
    mean1 = s1_ref[...] * inv_cnt
    var1 = q1_ref[...] * inv_cnt - mean1 * mean1
    a1 = g1_ref[...] / jnp.sqrt(var1 + 1e-5)
    c1 = be1_ref[...] - mean1 * a1
    z = jnp.maximum(y1_ref[0] * a1 + c1, 0.0)
    out_ref[0] = z.T


@jax.jit
def _run(xyz1, xyz2, points1, points2, W0, b0, g0, be0, W1, b1, g1, be1):
    b, n, _ = xyz1.shape
    m = xyz2.shape[1]
    c1 = points1.shape[2]
    c2 = points2.shape[2]
    cin = c1 + c2
    co0 = W0.shape[0]
    co1 = W1.shape[0]
    f32 = jnp.float32

    tn = min(256, n)
    grid = (b, n // tn)

    xyz2t = jnp.transpose(xyz2, (0, 2, 1))
    w0t = W0.T
    w1t = W1.T
    b0r = b0.reshape(1, co0)
    b1r = b1.reshape(1, co1)

    idx3, wts3 = pl.pallas_call(
        functools.partial(_knn_body, m=m),
        grid=grid,
        in_specs=[
            pl.BlockSpec((1, tn, 3), lambda bi, ti: (bi, ti, 0)),
            pl.BlockSpec((1, 3, m), lambda bi, ti: (bi, 0, 0)),
        ],
        out_specs=[
            pl.BlockSpec((1, 3, tn), lambda bi, ti: (bi, 0, ti)),
            pl.BlockSpec((1, tn, 3), lambda bi, ti: (bi, ti, 0)),
        ],
        out_shape=[
            jax.ShapeDtypeStruct((b, 3, n), jnp.int32),
            jax.ShapeDtypeStruct((b, n, 3), f32),
        ],
    )(xyz1, xyz2t)

    idx_flat = idx3.reshape(3 * b * n)
    table = points2.reshape(b * m, c2)
    gathered = _make_sc_gather(3 * b * n, c2, 32)(idx_flat, table)
    g4 = gathered.reshape(b, 3, n, c2)

    y0p = pl.pallas_call(
        _y0p_body,
        grid=grid,
        in_specs=[
            pl.BlockSpec((1, tn, c1), lambda bi, ti: (bi, ti, 0)),
            pl.BlockSpec((c1, co0), lambda bi, ti: (0, 0)),
            pl.BlockSpec((1, co0), lambda bi, ti: (0, 0)),
        ],
        out_specs=pl.BlockSpec((1, tn, co0), lambda bi, ti: (bi, ti, 0)),
        out_shape=jax.ShapeDtypeStruct((b, n, co0), f32),
    )(points1, w0t[c2:, :], b0r)

    y0, s0, q0 = pl.pallas_call(
        _interp_mlp0_body,
        grid=grid,
        in_specs=[
            pl.BlockSpec((1, 3, tn, c2), lambda bi, ti: (bi, 0, ti, 0)),
            pl.BlockSpec((1, tn, 3), lambda bi, ti: (bi, ti, 0)),
            pl.BlockSpec((1, tn, co0), lambda bi, ti: (bi, ti, 0)),
            pl.BlockSpec((c2, co0), lambda bi, ti: (0, 0)),
        ],
        out_specs=[
            pl.BlockSpec((1, tn, co0), lambda bi, ti: (bi, ti, 0)),
            pl.BlockSpec((1, co0), lambda bi, ti: (0, 0)),
            pl.BlockSpec((1, co0), lambda bi, ti: (0, 0)),
        ],
        out_shape=[
            jax.ShapeDtypeStruct((b, n, co0), f32),
            jax.ShapeDtypeStruct((1, co0), f32),
            jax.ShapeDtypeStruct((1, co0), f32),
        ],
    )(g4, wts3, y0p, w0t[:c2, :])

    inv_cnt = 1.0 / float(b * n)
    g0r = g0.reshape(1, co0)
    be0r = be0.reshape(1, co0)
    g1r = g1.reshape(1, co1)
    be1r = be1.reshape(1, co1)

    y1, s1, q1 = pl.pallas_call(
        functools.partial(_mlp1_body, inv_cnt=inv_cnt),
        grid=grid,
        in_specs=[
            pl.BlockSpec((1, tn, co0), lambda bi, ti: (bi, ti, 0)),
            pl.BlockSpec((1, co0), lambda bi, ti: (0, 0)),
            pl.BlockSpec((1, co0), lambda bi, ti: (0, 0)),
            pl.BlockSpec((1, co0), lambda bi, ti: (0, 0)),
            pl.BlockSpec((1, co0), lambda bi, ti: (0, 0)),
            pl.BlockSpec((co0, co1), lambda bi, ti: (0, 0)),
            pl.BlockSpec((1, co1), lambda bi, ti: (0, 0)),
        ],
        out_specs=[
            pl.BlockSpec((1, tn, co1), lambda bi, ti: (bi, ti, 0)),
            pl.BlockSpec((1, co1), lambda bi, ti: (0, 0)),
            pl.BlockSpec((1, co1), lambda bi, ti: (0, 0)),
        ],
        out_shape=[
            jax.ShapeDtypeStruct((b, n, co1), f32),
            jax.ShapeDtypeStruct((1, co1), f32),
            jax.ShapeDtypeStruct((1, co1), f32),
        ],
    )(y0, s0, q0, g0r, be0r, w1t, b1r)

    out = pl.pallas_call(
        functools.partial(_finalize_body, inv_cnt=inv_cnt),
        grid=grid,
        in_specs=[
            pl.BlockSpec((1, tn, co1), lambda bi, ti: (bi, ti, 0)),
            pl.BlockSpec((1, co1), lambda bi, ti: (0, 0)),
            pl.BlockSpec((1, co1), lambda bi, ti: (0, 0)),
            pl.BlockSpec((1, co1), lambda bi, ti: (0, 0)),
            pl.BlockSpec((1, co1), lambda bi, ti: (0, 0)),
        ],
        out_specs=pl.BlockSpec((1, co1, tn), lambda bi, ti: (bi, 0, ti)),
        out_shape=jax.ShapeDtypeStruct((b, co1, n), f32),
    )(y1, s1, q1, g1r, be1r)

    return out


def kernel(xyz1, xyz2, points1, points2, W0, b0, g0, be0, W1, b1, g1, be1):
    return _run(xyz1, xyz2, points1, points2, W0, b0, g0, be0,
                W1, b1, g1, be1)


# speedup vs baseline: 1.1209x; 1.1209x over previous
import functools

import jax
import jax.numpy as jnp
from jax import lax
from jax.experimental import pallas as pl
from jax.experimental.pallas import tpu as pltpu
from jax.experimental.pallas import tpu_sc as plsc


def _knn_body(x1_ref, x2t_ref, idx_ref, wts_ref, *, m):
    bi = pl.program_id(0)

    x1 = x1_ref[0]
    x2t = x2t_ref[0]
    dot = lax.dot_general(x1, x2t, (((1,), (0,)), ((), ())),
                          preferred_element_type=jnp.float32)
    x1sq = jnp.sum(x1 * x1, axis=1, keepdims=True)
    x2sq = jnp.sum(x2t * x2t, axis=0, keepdims=True)
    d2 = x1sq + x2sq - 2.0 * dot

    tn = d2.shape[0]
    iota = lax.broadcasted_iota(jnp.int32, (tn, m), 1).astype(jnp.float32)
    fm = jnp.float32(m)
    work = d2
    vals = []
    idxs = []
    for k in range(3):
        mv = jnp.min(work, axis=1, keepdims=True)
        mi = jnp.min(jnp.where(work == mv, iota, fm), axis=1, keepdims=True)
        idxs.append(mi)
        vals.append(mv)
        if k < 2:
            work = jnp.where(iota == mi, jnp.inf, work)

    rs = [1.0 / jnp.clip(v, 0.0, 1e-10) for v in vals]
    norm = rs[0] + rs[1] + rs[2]
    wts_ref[0] = jnp.concatenate(
        [rs[0] / norm, rs[1] / norm, rs[2] / norm], axis=1)
    gbase = (bi * m).astype(jnp.float32)
    idx_ref[0] = jnp.concatenate(
        [(g + gbase).astype(jnp.int32).T for g in idxs], axis=0)


def _make_sc_gather(n_rows, d, n_workers):
    mesh = plsc.VectorSubcoreMesh(core_axis_name="c", subcore_axis_name="s")
    rpw = n_rows // n_workers
    blk = 1024
    grp = 128
    nblk = rpw // blk

    @functools.partial(
        pl.kernel, mesh=mesh,
        out_type=jax.ShapeDtypeStruct((n_rows, d), jnp.float32),
        compiler_params=pltpu.CompilerParams(use_tc_tiling_on_sc=False),
        scratch_types=[
            pltpu.VMEM((blk,), jnp.int32),
            pltpu.VMEM((blk, d), jnp.float32),
            pltpu.SemaphoreType.DMA,
        ],
    )
    def gather(idx_hbm, table_hbm, out_hbm, idx_v, rows_v, sem):
        wid = lax.axis_index("s") * 2 + lax.axis_index("c")
        base = wid * rpw

        def body(i, carry):
            off = base + i * blk
            pltpu.sync_copy(idx_hbm.at[pl.ds(off, blk)], idx_v)
            copies = [
                pltpu.async_copy(table_hbm.at[idx_v.at[pl.ds(j * grp, grp)]],
                                 rows_v.at[pl.ds(j * grp, grp)], sem)
                for j in range(blk // grp)
            ]
            for cp in copies:
                cp.wait()
            pltpu.sync_copy(rows_v, out_hbm.at[pl.ds(off, blk)])
            return carry

        lax.fori_loop(0, nblk, body, 0)

    return gather


def _interp_mlp0_body(g_ref, w_ref, p1_ref, w0t_ref, b0_ref,
                      y0_ref, s_ref, q_ref, *, c2):
    bi = pl.program_id(0)
    ti = pl.program_id(1)
    w = w_ref[0]
    interp = (w[:, 0:1] * g_ref[0, 0]
              + w[:, 1:2] * g_ref[0, 1]
              + w[:, 2:3] * g_ref[0, 2])
    y0 = (lax.dot_general(interp, w0t_ref[:c2, :], (((1,), (0,)), ((), ())),
                          preferred_element_type=jnp.float32)
          + lax.dot_general(p1_ref[0], w0t_ref[c2:, :],
                            (((1,), (0,)), ((), ())),
                            preferred_element_type=jnp.float32)
          + b0_ref[...])
    y0_ref[0] = y0

    @pl.when(jnp.logical_and(bi == 0, ti == 0))
    def _init():
        s_ref[...] = jnp.zeros_like(s_ref)
        q_ref[...] = jnp.zeros_like(q_ref)

    s_ref[...] += jnp.sum(y0, axis=0, keepdims=True)
    q_ref[...] += jnp.sum(y0 * y0, axis=0, keepdims=True)


def _mlp1_body(y0_ref, s0_ref, q0_ref, g0_ref, be0_ref, w1t_ref, b1_ref,
               y1_ref, s_ref, q_ref, *, inv_cnt):
    bi = pl.program_id(0)
    ti = pl.program_id(1)
    mean0 = s0_ref[...] * inv_cnt
    var0 = q0_ref[...] * inv_cnt - mean0 * mean0
    a0 = g0_ref[...] / jnp.sqrt(var0 + 1e-5)
    c0 = be0_ref[...] - mean0 * a0
    h = jnp.maximum(y0_ref[0] * a0 + c0, 0.0)
    y1 = lax.dot_general(h, w1t_ref[...], (((1,), (0,)), ((), ())),
                         preferred_element_type=jnp.float32) + b1_ref[...]
    y1_ref[0] = y1

    @pl.when(jnp.logical_and(bi == 0, ti == 0))
    def _init():
        s_ref[...] = jnp.zeros_like(s_ref)
        q_ref[...] = jnp.zeros_like(q_ref)

    s_ref[...] += jnp.sum(y1, axis=0, keepdims=True)
    q_ref[...] += jnp.sum(y1 * y1, axis=0, keepdims=True)


def _finalize_body(y1_ref, s1_ref, q1_ref, g1_ref, be1_ref, out_ref, *,
                   inv_cnt):
    mean1 = s1_ref[...] * inv_cnt
    var1 = q1_ref[...] * inv_cnt - mean1 * mean1
    a1 = g1_ref[...] / jnp.sqrt(var1 + 1e-5)
    c1 = be1_ref[...] - mean1 * a1
    z = jnp.maximum(y1_ref[0] * a1 + c1, 0.0)
    out_ref[0] = z.T


@jax.jit
def _run(xyz1, xyz2, points1, points2, W0, b0, g0, be0, W1, b1, g1, be1):
    b, n, _ = xyz1.shape
    m = xyz2.shape[1]
    c1 = points1.shape[2]
    c2 = points2.shape[2]
    cin = c1 + c2
    co0 = W0.shape[0]
    co1 = W1.shape[0]
    f32 = jnp.float32

    tn = min(256, n)
    grid = (b, n // tn)

    xyz2t = jnp.transpose(xyz2, (0, 2, 1))
    w0t = W0.T
    w1t = W1.T
    b0r = b0.reshape(1, co0)
    b1r = b1.reshape(1, co1)

    idx3, wts3 = pl.pallas_call(
        functools.partial(_knn_body, m=m),
        grid=grid,
        in_specs=[
            pl.BlockSpec((1, tn, 3), lambda bi, ti: (bi, ti, 0)),
            pl.BlockSpec((1, 3, m), lambda bi, ti: (bi, 0, 0)),
        ],
        out_specs=[
            pl.BlockSpec((1, 3, tn), lambda bi, ti: (bi, 0, ti)),
            pl.BlockSpec((1, tn, 3), lambda bi, ti: (bi, ti, 0)),
        ],
        out_shape=[
            jax.ShapeDtypeStruct((b, 3, n), jnp.int32),
            jax.ShapeDtypeStruct((b, n, 3), f32),
        ],
    )(xyz1, xyz2t)

    idx_flat = idx3.reshape(3 * b * n)
    table = points2.reshape(b * m, c2)
    gathered = _make_sc_gather(3 * b * n, c2, 32)(idx_flat, table)
    g4 = gathered.reshape(b, 3, n, c2)

    y0, s0, q0 = pl.pallas_call(
        functools.partial(_interp_mlp0_body, c2=c2),
        grid=grid,
        in_specs=[
            pl.BlockSpec((1, 3, tn, c2), lambda bi, ti: (bi, 0, ti, 0)),
            pl.BlockSpec((1, tn, 3), lambda bi, ti: (bi, ti, 0)),
            pl.BlockSpec((1, tn, c1), lambda bi, ti: (bi, ti, 0)),
            pl.BlockSpec((cin, co0), lambda bi, ti: (0, 0)),
            pl.BlockSpec((1, co0), lambda bi, ti: (0, 0)),
        ],
        out_specs=[
            pl.BlockSpec((1, tn, co0), lambda bi, ti: (bi, ti, 0)),
            pl.BlockSpec((1, co0), lambda bi, ti: (0, 0)),
            pl.BlockSpec((1, co0), lambda bi, ti: (0, 0)),
        ],
        out_shape=[
            jax.ShapeDtypeStruct((b, n, co0), f32),
            jax.ShapeDtypeStruct((1, co0), f32),
            jax.ShapeDtypeStruct((1, co0), f32),
        ],
    )(g4, wts3, points1, w0t, b0r)

    inv_cnt = 1.0 / float(b * n)
    g0r = g0.reshape(1, co0)
    be0r = be0.reshape(1, co0)
    g1r = g1.reshape(1, co1)
    be1r = be1.reshape(1, co1)

    y1, s1, q1 = pl.pallas_call(
        functools.partial(_mlp1_body, inv_cnt=inv_cnt),
        grid=grid,
        in_specs=[
            pl.BlockSpec((1, tn, co0), lambda bi, ti: (bi, ti, 0)),
            pl.BlockSpec((1, co0), lambda bi, ti: (0, 0)),
            pl.BlockSpec((1, co0), lambda bi, ti: (0, 0)),
            pl.BlockSpec((1, co0), lambda bi, ti: (0, 0)),
            pl.BlockSpec((1, co0), lambda bi, ti: (0, 0)),
            pl.BlockSpec((co0, co1), lambda bi, ti: (0, 0)),
            pl.BlockSpec((1, co1), lambda bi, ti: (0, 0)),
        ],
        out_specs=[
            pl.BlockSpec((1, tn, co1), lambda bi, ti: (bi, ti, 0)),
            pl.BlockSpec((1, co1), lambda bi, ti: (0, 0)),
            pl.BlockSpec((1, co1), lambda bi, ti: (0, 0)),
        ],
        out_shape=[
            jax.ShapeDtypeStruct((b, n, co1), f32),
            jax.ShapeDtypeStruct((1, co1), f32),
            jax.ShapeDtypeStruct((1, co1), f32),
        ],
    )(y0, s0, q0, g0r, be0r, w1t, b1r)

    out = pl.pallas_call(
        functools.partial(_finalize_body, inv_cnt=inv_cnt),
        grid=grid,
        in_specs=[
            pl.BlockSpec((1, tn, co1), lambda bi, ti: (bi, ti, 0)),
            pl.BlockSpec((1, co1), lambda bi, ti: (0, 0)),
            pl.BlockSpec((1, co1), lambda bi, ti: (0, 0)),
            pl.BlockSpec((1, co1), lambda bi, ti: (0, 0)),
            pl.BlockSpec((1, co1), lambda bi, ti: (0, 0)),
        ],
        out_specs=pl.BlockSpec((1, co1, tn), lambda bi, ti: (bi, 0, ti)),
        out_shape=jax.ShapeDtypeStruct((b, co1, n), f32),
    )(y1, s1, q1, g1r, be1r)

    return out


def kernel(xyz1, xyz2, points1, points2, W0, b0, g0, be0, W1, b1, g1, be1):
    return _run(xyz1, xyz2, points1, points2, W0, b0, g0, be0,
                W1, b1, g1, be1)


# speedup vs baseline: 1.1870x; 1.0590x over previous
import functools

import jax
import jax.numpy as jnp
from jax import lax
from jax.experimental import pallas as pl
from jax.experimental.pallas import tpu as pltpu


def _mega_body(x1_ref, x2t_ref, p1_ref, p2_ref, w0t_ref, b0_ref, w1t_ref,
               b1_ref, g0_ref, be0_ref, g1_ref, be1_ref, out_ref,
               act_ref, s0_ref, q0_ref, s1_ref, q1_ref,
               *, m, c2, n, tn, inv_cnt):
    ph = pl.program_id(0)
    bi = pl.program_id(1)
    ti = pl.program_id(2)
    off = bi * n + ti * tn

    @pl.when(ph == 0)
    def _phase0():
        x1 = x1_ref[0]
        x2t = x2t_ref[0]
        dot = lax.dot_general(x1, x2t, (((1,), (0,)), ((), ())),
                              preferred_element_type=jnp.float32)
        x1sq = jnp.sum(x1 * x1, axis=1, keepdims=True)
        x2sq = jnp.sum(x2t * x2t, axis=0, keepdims=True)
        d2 = x1sq + x2sq - 2.0 * dot

        iota = lax.broadcasted_iota(jnp.int32, (tn, m), 1).astype(jnp.float32)
        fm = jnp.float32(m)
        work = d2
        vals = []
        masks = []
        for k in range(3):
            mv = jnp.min(work, axis=1, keepdims=True)
            mi = jnp.min(jnp.where(work == mv, iota, fm), axis=1,
                         keepdims=True)
            sel = iota == mi
            masks.append(sel)
            vals.append(mv)
            if k < 2:
                work = jnp.where(sel, jnp.inf, work)

        rs = [1.0 / jnp.clip(v, 0.0, 1e-10) for v in vals]
        norm = rs[0] + rs[1] + rs[2]
        zero = jnp.zeros_like(d2)
        wsp = jnp.where(masks[0], rs[0] / norm,
                        jnp.where(masks[1], rs[1] / norm,
                                  jnp.where(masks[2], rs[2] / norm, zero)))

        interp = lax.dot_general(wsp, p2_ref[0], (((1,), (0,)), ((), ())),
                                 preferred_element_type=jnp.float32)

        y0 = (lax.dot_general(interp, w0t_ref[0:c2, :],
                              (((1,), (0,)), ((), ())),
                              preferred_element_type=jnp.float32)
              + lax.dot_general(p1_ref[0], w0t_ref[c2:, :],
                                (((1,), (0,)), ((), ())),
                                preferred_element_type=jnp.float32)
              + b0_ref[...])
        act_ref[pl.ds(off, tn), :] = y0

        @pl.when(jnp.logical_and(bi == 0, ti == 0))
        def _init0():
            s0_ref[...] = jnp.zeros_like(s0_ref)
            q0_ref[...] = jnp.zeros_like(q0_ref)

        s0_ref[...] += jnp.sum(y0, axis=0, keepdims=True)
        q0_ref[...] += jnp.sum(y0 * y0, axis=0, keepdims=True)

    @pl.when(ph == 1)
    def _phase1():
        mean0 = s0_ref[...] * inv_cnt
        var0 = q0_ref[...] * inv_cnt - mean0 * mean0
        a0 = g0_ref[...] / jnp.sqrt(var0 + 1e-5)
        c0 = be0_ref[...] - mean0 * a0
        h = jnp.maximum(act_ref[pl.ds(off, tn), :] * a0 + c0, 0.0)
        y1 = lax.dot_general(h, w1t_ref[...], (((1,), (0,)), ((), ())),
                             preferred_element_type=jnp.float32) + b1_ref[...]
        act_ref[pl.ds(off, tn), :] = y1

        @pl.when(jnp.logical_and(bi == 0, ti == 0))
        def _init1():
            s1_ref[...] = jnp.zeros_like(s1_ref)
            q1_ref[...] = jnp.zeros_like(q1_ref)

        s1_ref[...] += jnp.sum(y1, axis=0, keepdims=True)
        q1_ref[...] += jnp.sum(y1 * y1, axis=0, keepdims=True)

    @pl.when(ph == 2)
    def _phase2():
        mean1 = s1_ref[...] * inv_cnt
        var1 = q1_ref[...] * inv_cnt - mean1 * mean1
        a1 = g1_ref[...] / jnp.sqrt(var1 + 1e-5)
        c1 = be1_ref[...] - mean1 * a1
        z = jnp.maximum(act_ref[pl.ds(off, tn), :] * a1 + c1, 0.0)
        out_ref[0] = z.T


@jax.jit
def _run(xyz1, xyz2, points1, points2, W0, b0, g0, be0, W1, b1, g1, be1):
    b, n, _ = xyz1.shape
    m = xyz2.shape[1]
    c1 = points1.shape[2]
    c2 = points2.shape[2]
    cin = c1 + c2
    co0 = W0.shape[0]
    co1 = W1.shape[0]
    f32 = jnp.float32

    tn = min(256, n)
    grid = (3, b, n // tn)

    xyz2t = jnp.transpose(xyz2, (0, 2, 1))
    w0t = W0.T
    w1t = W1.T
    b0r = b0.reshape(1, co0)
    b1r = b1.reshape(1, co1)
    g0r = g0.reshape(1, co0)
    be0r = be0.reshape(1, co0)
    g1r = g1.reshape(1, co1)
    be1r = be1.reshape(1, co1)

    body = functools.partial(_mega_body, m=m, c2=c2, n=n, tn=tn,
                             inv_cnt=1.0 / float(b * n))

    out = pl.pallas_call(
        body,
        grid=grid,
        in_specs=[
            pl.BlockSpec((1, tn, 3), lambda ph, bi, ti: (bi, ti, 0)),
            pl.BlockSpec((1, 3, m), lambda ph, bi, ti: (bi, 0, 0)),
            pl.BlockSpec((1, tn, c1), lambda ph, bi, ti: (bi, ti, 0)),
            pl.BlockSpec((1, m, c2), lambda ph, bi, ti: (bi, 0, 0)),
            pl.BlockSpec((cin, co0), lambda ph, bi, ti: (0, 0)),
            pl.BlockSpec((1, co0), lambda ph, bi, ti: (0, 0)),
            pl.BlockSpec((co0, co1), lambda ph, bi, ti: (0, 0)),
            pl.BlockSpec((1, co1), lambda ph, bi, ti: (0, 0)),
            pl.BlockSpec((1, co0), lambda ph, bi, ti: (0, 0)),
            pl.BlockSpec((1, co0), lambda ph, bi, ti: (0, 0)),
            pl.BlockSpec((1, co1), lambda ph, bi, ti: (0, 0)),
            pl.BlockSpec((1, co1), lambda ph, bi, ti: (0, 0)),
        ],
        out_specs=pl.BlockSpec((1, co1, tn), lambda ph, bi, ti: (bi, 0, ti)),
        out_shape=jax.ShapeDtypeStruct((b, co1, n), f32),
        scratch_shapes=[
            pltpu.VMEM((b * n, co0), f32),
            pltpu.VMEM((1, co0), f32),
            pltpu.VMEM((1, co0), f32),
            pltpu.VMEM((1, co1), f32),
            pltpu.VMEM((1, co1), f32),
        ],
        compiler_params=pltpu.CompilerParams(
            dimension_semantics=("arbitrary", "arbitrary", "arbitrary"),
            vmem_limit_bytes=100 * 1024 * 1024,
        ),
    )(xyz1, xyz2t, points1, points2, w0t, b0r, w1t, b1r,
      g0r, be0r, g1r, be1r)

    return out


def kernel(xyz1, xyz2, points1, points2, W0, b0, g0, be0, W1, b1, g1, be1):
    return _run(xyz1, xyz2, points1, points2, W0, b0, g0, be0,
                W1, b1, g1, be1)


# speedup vs baseline: 1.4241x; 1.1997x over previous
import functools

import jax
import jax.numpy as jnp
from jax import lax
from jax.experimental import pallas as pl
from jax.experimental.pallas import tpu as pltpu
from jax.experimental.pallas import tpu_sc as plsc


def _knn_body(x1_ref, x2t_ref, idx_ref, wts_ref, *, m):
    bi = pl.program_id(0)

    x1 = x1_ref[0]
    x2t = x2t_ref[0]
    dot = lax.dot_general(x1, x2t, (((1,), (0,)), ((), ())),
                          preferred_element_type=jnp.float32)
    x1sq = jnp.sum(x1 * x1, axis=1, keepdims=True)
    x2sq = jnp.sum(x2t * x2t, axis=0, keepdims=True)
    d2 = x1sq + x2sq - 2.0 * dot

    tn = d2.shape[0]
    iota = lax.broadcasted_iota(jnp.int32, (tn, m), 1).astype(jnp.float32)
    fm = jnp.float32(m)
    work = d2
    vals = []
    idxs = []
    for k in range(3):
        mv = jnp.min(work, axis=1, keepdims=True)
        mi = jnp.min(jnp.where(work == mv, iota, fm), axis=1, keepdims=True)
        idxs.append(mi)
        vals.append(mv)
        if k < 2:
            work = jnp.where(iota == mi, jnp.inf, work)

    rs = [1.0 / jnp.clip(v, 0.0, 1e-10) for v in vals]
    norm = rs[0] + rs[1] + rs[2]
    wts_ref[0] = jnp.concatenate(
        [rs[0] / norm, rs[1] / norm, rs[2] / norm], axis=1)
    gbase = (bi * m).astype(jnp.float32)
    idx_ref[0] = jnp.concatenate(
        [(g + gbase).astype(jnp.int32).T for g in idxs], axis=0)


def _make_sc_gather(n_rows, d, n_workers):
    mesh = plsc.VectorSubcoreMesh(core_axis_name="c", subcore_axis_name="s")
    rpw = n_rows // n_workers
    blk = 1024
    grp = 128
    nblk = rpw // blk

    @functools.partial(
        pl.kernel, mesh=mesh,
        out_type=jax.ShapeDtypeStruct((n_rows, d), jnp.float32),
        compiler_params=pltpu.CompilerParams(use_tc_tiling_on_sc=False),
        scratch_types=[
            pltpu.VMEM((blk,), jnp.int32),
            pltpu.VMEM((blk, d), jnp.float32),
            pltpu.SemaphoreType.DMA,
        ],
    )
    def gather(idx_hbm, table_hbm, out_hbm, idx_v, rows_v, sem):
        wid = lax.axis_index("s") * 2 + lax.axis_index("c")
        base = wid * rpw

        def body(i, carry):
            off = base + i * blk
            pltpu.sync_copy(idx_hbm.at[pl.ds(off, blk)], idx_v)
            copies = [
                pltpu.async_copy(table_hbm.at[idx_v.at[pl.ds(j * grp, grp)]],
                                 rows_v.at[pl.ds(j * grp, grp)], sem)
                for j in range(blk // grp)
            ]
            for cp in copies:
                cp.wait()
            pltpu.sync_copy(rows_v, out_hbm.at[pl.ds(off, blk)])
            return carry

        lax.fori_loop(0, nblk, body, 0)

    return gather


def _interp_mlp0_body(g_ref, w_ref, p1_ref, w0t_ref, b0_ref,
                      y0_ref, s_ref, q_ref, *, c2):
    bi = pl.program_id(0)
    ti = pl.program_id(1)
    w = w_ref[0]
    interp = (w[:, 0:1] * g_ref[0, 0]
              + w[:, 1:2] * g_ref[0, 1]
              + w[:, 2:3] * g_ref[0, 2])
    y0 = (lax.dot_general(interp, w0t_ref[:c2, :], (((1,), (0,)), ((), ())),
                          preferred_element_type=jnp.float32)
          + lax.dot_general(p1_ref[0], w0t_ref[c2:, :],
                            (((1,), (0,)), ((), ())),
                            preferred_element_type=jnp.float32)
          + b0_ref[...])
    y0_ref[0] = y0

    @pl.when(jnp.logical_and(bi == 0, ti == 0))
    def _init():
        s_ref[...] = jnp.zeros_like(s_ref)
        q_ref[...] = jnp.zeros_like(q_ref)

    s_ref[...] += jnp.sum(y0, axis=0, keepdims=True)
    q_ref[...] += jnp.sum(y0 * y0, axis=0, keepdims=True)


def _mlp1_body(y0_ref, s0_ref, q0_ref, g0_ref, be0_ref, w1t_ref, b1_ref,
               y1_ref, s_ref, q_ref, *, inv_cnt):
    bi = pl.program_id(0)
    ti = pl.program_id(1)
    mean0 = s0_ref[...] * inv_cnt
    var0 = q0_ref[...] * inv_cnt - mean0 * mean0
    a0 = g0_ref[...] / jnp.sqrt(var0 + 1e-5)
    c0 = be0_ref[...] - mean0 * a0
    h = jnp.maximum(y0_ref[0] * a0 + c0, 0.0)
    y1 = lax.dot_general(h, w1t_ref[...], (((1,), (0,)), ((), ())),
                         preferred_element_type=jnp.float32) + b1_ref[...]
    y1_ref[0] = y1

    @pl.when(jnp.logical_and(bi == 0, ti == 0))
    def _init():
        s_ref[...] = jnp.zeros_like(s_ref)
        q_ref[...] = jnp.zeros_like(q_ref)

    s_ref[...] += jnp.sum(y1, axis=0, keepdims=True)
    q_ref[...] += jnp.sum(y1 * y1, axis=0, keepdims=True)


def _finalize_body(y1_ref, s1_ref, q1_ref, g1_ref, be1_ref, out_ref, *,
                   inv_cnt):
    mean1 = s1_ref[...] * inv_cnt
    var1 = q1_ref[...] * inv_cnt - mean1 * mean1
    a1 = g1_ref[...] / jnp.sqrt(var1 + 1e-5)
    c1 = be1_ref[...] - mean1 * a1
    z = jnp.maximum(y1_ref[0] * a1 + c1, 0.0)
    out_ref[0] = z.T


@jax.jit
def _run(xyz1, xyz2, points1, points2, W0, b0, g0, be0, W1, b1, g1, be1):
    b, n, _ = xyz1.shape
    m = xyz2.shape[1]
    c1 = points1.shape[2]
    c2 = points2.shape[2]
    cin = c1 + c2
    co0 = W0.shape[0]
    co1 = W1.shape[0]
    f32 = jnp.float32

    tn = min(512, n)
    grid = (b, n // tn)

    xyz2t = jnp.transpose(xyz2, (0, 2, 1))
    w0t = W0.T
    w1t = W1.T
    b0r = b0.reshape(1, co0)
    b1r = b1.reshape(1, co1)

    idx3, wts3 = pl.pallas_call(
        functools.partial(_knn_body, m=m),
        grid=grid,
        in_specs=[
            pl.BlockSpec((1, tn, 3), lambda bi, ti: (bi, ti, 0)),
            pl.BlockSpec((1, 3, m), lambda bi, ti: (bi, 0, 0)),
        ],
        out_specs=[
            pl.BlockSpec((1, 3, tn), lambda bi, ti: (bi, 0, ti)),
            pl.BlockSpec((1, tn, 3), lambda bi, ti: (bi, ti, 0)),
        ],
        out_shape=[
            jax.ShapeDtypeStruct((b, 3, n), jnp.int32),
            jax.ShapeDtypeStruct((b, n, 3), f32),
        ],
    )(xyz1, xyz2t)

    idx_flat = idx3.reshape(3 * b * n)
    table = points2.reshape(b * m, c2)
    gathered = _make_sc_gather(3 * b * n, c2, 32)(idx_flat, table)
    g4 = gathered.reshape(b, 3, n, c2)

    y0, s0, q0 = pl.pallas_call(
        functools.partial(_interp_mlp0_body, c2=c2),
        grid=grid,
        in_specs=[
            pl.BlockSpec((1, 3, tn, c2), lambda bi, ti: (bi, 0, ti, 0)),
            pl.BlockSpec((1, tn, 3), lambda bi, ti: (bi, ti, 0)),
            pl.BlockSpec((1, tn, c1), lambda bi, ti: (bi, ti, 0)),
            pl.BlockSpec((cin, co0), lambda bi, ti: (0, 0)),
            pl.BlockSpec((1, co0), lambda bi, ti: (0, 0)),
        ],
        out_specs=[
            pl.BlockSpec((1, tn, co0), lambda bi, ti: (bi, ti, 0)),
            pl.BlockSpec((1, co0), lambda bi, ti: (0, 0)),
            pl.BlockSpec((1, co0), lambda bi, ti: (0, 0)),
        ],
        out_shape=[
            jax.ShapeDtypeStruct((b, n, co0), f32),
            jax.ShapeDtypeStruct((1, co0), f32),
            jax.ShapeDtypeStruct((1, co0), f32),
        ],
    )(g4, wts3, points1, w0t, b0r)

    inv_cnt = 1.0 / float(b * n)
    g0r = g0.reshape(1, co0)
    be0r = be0.reshape(1, co0)
    g1r = g1.reshape(1, co1)
    be1r = be1.reshape(1, co1)

    y1, s1, q1 = pl.pallas_call(
        functools.partial(_mlp1_body, inv_cnt=inv_cnt),
        grid=grid,
        in_specs=[
            pl.BlockSpec((1, tn, co0), lambda bi, ti: (bi, ti, 0)),
            pl.BlockSpec((1, co0), lambda bi, ti: (0, 0)),
            pl.BlockSpec((1, co0), lambda bi, ti: (0, 0)),
            pl.BlockSpec((1, co0), lambda bi, ti: (0, 0)),
            pl.BlockSpec((1, co0), lambda bi, ti: (0, 0)),
            pl.BlockSpec((co0, co1), lambda bi, ti: (0, 0)),
            pl.BlockSpec((1, co1), lambda bi, ti: (0, 0)),
        ],
        out_specs=[
            pl.BlockSpec((1, tn, co1), lambda bi, ti: (bi, ti, 0)),
            pl.BlockSpec((1, co1), lambda bi, ti: (0, 0)),
            pl.BlockSpec((1, co1), lambda bi, ti: (0, 0)),
        ],
        out_shape=[
            jax.ShapeDtypeStruct((b, n, co1), f32),
            jax.ShapeDtypeStruct((1, co1), f32),
            jax.ShapeDtypeStruct((1, co1), f32),
        ],
    )(y0, s0, q0, g0r, be0r, w1t, b1r)

    out = pl.pallas_call(
        functools.partial(_finalize_body, inv_cnt=inv_cnt),
        grid=grid,
        in_specs=[
            pl.BlockSpec((1, tn, co1), lambda bi, ti: (bi, ti, 0)),
            pl.BlockSpec((1, co1), lambda bi, ti: (0, 0)),
            pl.BlockSpec((1, co1), lambda bi, ti: (0, 0)),
            pl.BlockSpec((1, co1), lambda bi, ti: (0, 0)),
            pl.BlockSpec((1, co1), lambda bi, ti: (0, 0)),
        ],
        out_specs=pl.BlockSpec((1, co1, tn), lambda bi, ti: (bi, 0, ti)),
        out_shape=jax.ShapeDtypeStruct((b, co1, n), f32),
    )(y1, s1, q1, g1r, be1r)

    return out


def kernel(xyz1, xyz2, points1, points2, W0, b0, g0, be0, W1, b1, g1, be1):
    return _run(xyz1, xyz2, points1, points2, W0, b0, g0, be0,
                W1, b1, g1, be1)


# speedup vs baseline: 1.6635x; 1.1681x over previous
import functools

import jax
import jax.numpy as jnp
from jax import lax
from jax.experimental import pallas as pl
from jax.experimental.pallas import tpu as pltpu
from jax.experimental.pallas import tpu_sc as plsc


def _knn_body(x1_ref, x2t_ref, idx_ref, wts_ref, *, m):
    bi = pl.program_id(0)

    x1 = x1_ref[0]
    x2t = x2t_ref[0]
    dot = lax.dot_general(x1, x2t, (((1,), (0,)), ((), ())),
                          preferred_element_type=jnp.float32)
    x1sq = jnp.sum(x1 * x1, axis=1, keepdims=True)
    x2sq = jnp.sum(x2t * x2t, axis=0, keepdims=True)
    d2 = x1sq + x2sq - 2.0 * dot

    tn = d2.shape[0]
    iota = lax.broadcasted_iota(jnp.int32, (tn, m), 1).astype(jnp.float32)
    fm = jnp.float32(m)
    work = d2
    vals = []
    idxs = []
    for k in range(3):
        mv = jnp.min(work, axis=1, keepdims=True)
        mi = jnp.min(jnp.where(work == mv, iota, fm), axis=1, keepdims=True)
        idxs.append(mi)
        vals.append(mv)
        if k < 2:
            work = jnp.where(iota == mi, jnp.inf, work)

    rs = [1.0 / jnp.clip(v, 0.0, 1e-10) for v in vals]
    norm = rs[0] + rs[1] + rs[2]
    wts_ref[0] = jnp.concatenate(
        [rs[0] / norm, rs[1] / norm, rs[2] / norm], axis=1)
    gbase = (bi * m).astype(jnp.float32)
    idx_ref[0] = jnp.concatenate(
        [(g + gbase).astype(jnp.int32).T for g in idxs], axis=0)


def _make_sc_gather(n_rows, d, n_workers):
    mesh = plsc.VectorSubcoreMesh(core_axis_name="c", subcore_axis_name="s")
    rpw = n_rows // n_workers
    blk = 1024
    grp = 128
    nblk = rpw // blk

    @functools.partial(
        pl.kernel, mesh=mesh,
        out_type=jax.ShapeDtypeStruct((n_rows, d), jnp.float32),
        compiler_params=pltpu.CompilerParams(use_tc_tiling_on_sc=False),
        scratch_types=[
            pltpu.VMEM((blk,), jnp.int32),
            pltpu.VMEM((blk, d), jnp.float32),
            pltpu.SemaphoreType.DMA,
        ],
    )
    def gather(idx_hbm, table_hbm, out_hbm, idx_v, rows_v, sem):
        wid = lax.axis_index("s") * 2 + lax.axis_index("c")
        base = wid * rpw

        def body(i, carry):
            off = base + i * blk
            pltpu.sync_copy(idx_hbm.at[pl.ds(off, blk)], idx_v)
            copies = [
                pltpu.async_copy(table_hbm.at[idx_v.at[pl.ds(j * grp, grp)]],
                                 rows_v.at[pl.ds(j * grp, grp)], sem)
                for j in range(blk // grp)
            ]
            for cp in copies:
                cp.wait()
            pltpu.sync_copy(rows_v, out_hbm.at[pl.ds(off, blk)])
            return carry

        lax.fori_loop(0, nblk, body, 0)

    return gather


def _interp_mlp0_body(g_ref, w_ref, p1_ref, w0t_ref, b0_ref,
                      y0_ref, s_ref, q_ref, *, c2):
    bi = pl.program_id(0)
    ti = pl.program_id(1)
    w = w_ref[0]
    interp = (w[:, 0:1] * g_ref[0, 0]
              + w[:, 1:2] * g_ref[0, 1]
              + w[:, 2:3] * g_ref[0, 2])
    y0 = (lax.dot_general(interp, w0t_ref[:c2, :], (((1,), (0,)), ((), ())),
                          preferred_element_type=jnp.float32)
          + lax.dot_general(p1_ref[0], w0t_ref[c2:, :],
                            (((1,), (0,)), ((), ())),
                            preferred_element_type=jnp.float32)
          + b0_ref[...])
    y0_ref[0] = y0

    @pl.when(jnp.logical_and(bi == 0, ti == 0))
    def _init():
        s_ref[...] = jnp.zeros_like(s_ref)
        q_ref[...] = jnp.zeros_like(q_ref)

    s_ref[...] += jnp.sum(y0, axis=0, keepdims=True)
    q_ref[...] += jnp.sum(y0 * y0, axis=0, keepdims=True)


def _mlp1_body(y0_ref, s0_ref, q0_ref, g0_ref, be0_ref, w1t_ref, b1_ref,
               y1_ref, s_ref, q_ref, *, inv_cnt):
    bi = pl.program_id(0)
    ti = pl.program_id(1)
    mean0 = s0_ref[...] * inv_cnt
    var0 = q0_ref[...] * inv_cnt - mean0 * mean0
    a0 = g0_ref[...] / jnp.sqrt(var0 + 1e-5)
    c0 = be0_ref[...] - mean0 * a0
    h = jnp.maximum(y0_ref[0] * a0 + c0, 0.0)
    y1 = lax.dot_general(h, w1t_ref[...], (((1,), (0,)), ((), ())),
                         preferred_element_type=jnp.float32) + b1_ref[...]
    y1_ref[0] = y1

    @pl.when(jnp.logical_and(bi == 0, ti == 0))
    def _init():
        s_ref[...] = jnp.zeros_like(s_ref)
        q_ref[...] = jnp.zeros_like(q_ref)

    s_ref[...] += jnp.sum(y1, axis=0, keepdims=True)
    q_ref[...] += jnp.sum(y1 * y1, axis=0, keepdims=True)


def _finalize_body(y1_ref, s1_ref, q1_ref, g1_ref, be1_ref, out_ref, *,
                   inv_cnt):
    mean1 = s1_ref[...] * inv_cnt
    var1 = q1_ref[...] * inv_cnt - mean1 * mean1
    a1 = g1_ref[...] / jnp.sqrt(var1 + 1e-5)
    c1 = be1_ref[...] - mean1 * a1
    z = jnp.maximum(y1_ref[0] * a1 + c1, 0.0)
    out_ref[0] = z.T


@jax.jit
def _run(xyz1, xyz2, points1, points2, W0, b0, g0, be0, W1, b1, g1, be1):
    b, n, _ = xyz1.shape
    m = xyz2.shape[1]
    c1 = points1.shape[2]
    c2 = points2.shape[2]
    cin = c1 + c2
    co0 = W0.shape[0]
    co1 = W1.shape[0]
    f32 = jnp.float32

    tn = min(1024, n)
    grid = (b, n // tn)

    xyz2t = jnp.transpose(xyz2, (0, 2, 1))
    w0t = W0.T
    w1t = W1.T
    b0r = b0.reshape(1, co0)
    b1r = b1.reshape(1, co1)

    idx3, wts3 = pl.pallas_call(
        functools.partial(_knn_body, m=m),
        grid=grid,
        in_specs=[
            pl.BlockSpec((1, tn, 3), lambda bi, ti: (bi, ti, 0)),
            pl.BlockSpec((1, 3, m), lambda bi, ti: (bi, 0, 0)),
        ],
        out_specs=[
            pl.BlockSpec((1, 3, tn), lambda bi, ti: (bi, 0, ti)),
            pl.BlockSpec((1, tn, 3), lambda bi, ti: (bi, ti, 0)),
        ],
        out_shape=[
            jax.ShapeDtypeStruct((b, 3, n), jnp.int32),
            jax.ShapeDtypeStruct((b, n, 3), f32),
        ],
    )(xyz1, xyz2t)

    idx_flat = idx3.reshape(3 * b * n)
    table = points2.reshape(b * m, c2)
    gathered = _make_sc_gather(3 * b * n, c2, 32)(idx_flat, table)
    g4 = gathered.reshape(b, 3, n, c2)

    y0, s0, q0 = pl.pallas_call(
        functools.partial(_interp_mlp0_body, c2=c2),
        grid=grid,
        in_specs=[
            pl.BlockSpec((1, 3, tn, c2), lambda bi, ti: (bi, 0, ti, 0)),
            pl.BlockSpec((1, tn, 3), lambda bi, ti: (bi, ti, 0)),
            pl.BlockSpec((1, tn, c1), lambda bi, ti: (bi, ti, 0)),
            pl.BlockSpec((cin, co0), lambda bi, ti: (0, 0)),
            pl.BlockSpec((1, co0), lambda bi, ti: (0, 0)),
        ],
        out_specs=[
            pl.BlockSpec((1, tn, co0), lambda bi, ti: (bi, ti, 0)),
            pl.BlockSpec((1, co0), lambda bi, ti: (0, 0)),
            pl.BlockSpec((1, co0), lambda bi, ti: (0, 0)),
        ],
        out_shape=[
            jax.ShapeDtypeStruct((b, n, co0), f32),
            jax.ShapeDtypeStruct((1, co0), f32),
            jax.ShapeDtypeStruct((1, co0), f32),
        ],
    )(g4, wts3, points1, w0t, b0r)

    inv_cnt = 1.0 / float(b * n)
    g0r = g0.reshape(1, co0)
    be0r = be0.reshape(1, co0)
    g1r = g1.reshape(1, co1)
    be1r = be1.reshape(1, co1)

    y1, s1, q1 = pl.pallas_call(
        functools.partial(_mlp1_body, inv_cnt=inv_cnt),
        grid=grid,
        in_specs=[
            pl.BlockSpec((1, tn, co0), lambda bi, ti: (bi, ti, 0)),
            pl.BlockSpec((1, co0), lambda bi, ti: (0, 0)),
            pl.BlockSpec((1, co0), lambda bi, ti: (0, 0)),
            pl.BlockSpec((1, co0), lambda bi, ti: (0, 0)),
            pl.BlockSpec((1, co0), lambda bi, ti: (0, 0)),
            pl.BlockSpec((co0, co1), lambda bi, ti: (0, 0)),
            pl.BlockSpec((1, co1), lambda bi, ti: (0, 0)),
        ],
        out_specs=[
            pl.BlockSpec((1, tn, co1), lambda bi, ti: (bi, ti, 0)),
            pl.BlockSpec((1, co1), lambda bi, ti: (0, 0)),
            pl.BlockSpec((1, co1), lambda bi, ti: (0, 0)),
        ],
        out_shape=[
            jax.ShapeDtypeStruct((b, n, co1), f32),
            jax.ShapeDtypeStruct((1, co1), f32),
            jax.ShapeDtypeStruct((1, co1), f32),
        ],
    )(y0, s0, q0, g0r, be0r, w1t, b1r)

    out = pl.pallas_call(
        functools.partial(_finalize_body, inv_cnt=inv_cnt),
        grid=grid,
        in_specs=[
            pl.BlockSpec((1, tn, co1), lambda bi, ti: (bi, ti, 0)),
            pl.BlockSpec((1, co1), lambda bi, ti: (0, 0)),
            pl.BlockSpec((1, co1), lambda bi, ti: (0, 0)),
            pl.BlockSpec((1, co1), lambda bi, ti: (0, 0)),
            pl.BlockSpec((1, co1), lambda bi, ti: (0, 0)),
        ],
        out_specs=pl.BlockSpec((1, co1, tn), lambda bi, ti: (bi, 0, ti)),
        out_shape=jax.ShapeDtypeStruct((b, co1, n), f32),
    )(y1, s1, q1, g1r, be1r)

    return out


def kernel(xyz1, xyz2, points1, points2, W0, b0, g0, be0, W1, b1, g1, be1):
    return _run(xyz1, xyz2, points1, points2, W0, b0, g0, be0,
                W1, b1, g1, be1)


# speedup vs baseline: 1.8099x; 1.0880x over previous
import functools

import jax
import jax.numpy as jnp
from jax import lax
from jax.experimental import pallas as pl
from jax.experimental.pallas import tpu as pltpu
from jax.experimental.pallas import tpu_sc as plsc


def _knn_body(x1_ref, x2t_ref, idx_ref, wts_ref, *, m):
    bi = pl.program_id(0)

    x1 = x1_ref[0]
    x2t = x2t_ref[0]
    dot = lax.dot_general(x1, x2t, (((1,), (0,)), ((), ())),
                          preferred_element_type=jnp.float32)
    x1sq = jnp.sum(x1 * x1, axis=1, keepdims=True)
    x2sq = jnp.sum(x2t * x2t, axis=0, keepdims=True)
    d2 = x1sq + x2sq - 2.0 * dot

    tn = d2.shape[0]
    iota = lax.broadcasted_iota(jnp.int32, (tn, m), 1).astype(jnp.float32)
    fm = jnp.float32(m)
    work = d2
    vals = []
    idxs = []
    for k in range(3):
        mv = jnp.min(work, axis=1, keepdims=True)
        mi = jnp.min(jnp.where(work == mv, iota, fm), axis=1, keepdims=True)
        idxs.append(mi)
        vals.append(mv)
        if k < 2:
            work = jnp.where(iota == mi, jnp.inf, work)

    rs = [1.0 / jnp.clip(v, 0.0, 1e-10) for v in vals]
    norm = rs[0] + rs[1] + rs[2]
    wts_ref[0] = jnp.concatenate(
        [rs[0] / norm, rs[1] / norm, rs[2] / norm], axis=1)
    gbase = (bi * m).astype(jnp.float32)
    idx_ref[0] = jnp.concatenate(
        [(g + gbase).astype(jnp.int32).T for g in idxs], axis=0)


def _make_sc_gather(n_rows, d, n_workers):
    mesh = plsc.VectorSubcoreMesh(core_axis_name="c", subcore_axis_name="s")
    rpw = n_rows // n_workers
    blk = 1024
    grp = 128
    nblk = rpw // blk

    @functools.partial(
        pl.kernel, mesh=mesh,
        out_type=jax.ShapeDtypeStruct((n_rows, d), jnp.float32),
        compiler_params=pltpu.CompilerParams(use_tc_tiling_on_sc=False),
        scratch_types=[
            pltpu.VMEM((blk,), jnp.int32),
            pltpu.VMEM((blk, d), jnp.float32),
            pltpu.SemaphoreType.DMA,
        ],
    )
    def gather(idx_hbm, table_hbm, out_hbm, idx_v, rows_v, sem):
        wid = lax.axis_index("s") * 2 + lax.axis_index("c")
        base = wid * rpw

        def body(i, carry):
            off = base + i * blk
            pltpu.sync_copy(idx_hbm.at[pl.ds(off, blk)], idx_v)
            copies = [
                pltpu.async_copy(table_hbm.at[idx_v.at[pl.ds(j * grp, grp)]],
                                 rows_v.at[pl.ds(j * grp, grp)], sem)
                for j in range(blk // grp)
            ]
            for cp in copies:
                cp.wait()
            pltpu.sync_copy(rows_v, out_hbm.at[pl.ds(off, blk)])
            return carry

        lax.fori_loop(0, nblk, body, 0)

    return gather


def _interp_mlp0_body(g_ref, w_ref, p1_ref, w0t_ref, b0_ref,
                      y0_ref, s_ref, q_ref, *, c2):
    bi = pl.program_id(0)
    ti = pl.program_id(1)
    w = w_ref[0]
    interp = (w[:, 0:1] * g_ref[0, 0]
              + w[:, 1:2] * g_ref[0, 1]
              + w[:, 2:3] * g_ref[0, 2])
    y0 = (lax.dot_general(interp, w0t_ref[:c2, :], (((1,), (0,)), ((), ())),
                          preferred_element_type=jnp.float32)
          + lax.dot_general(p1_ref[0], w0t_ref[c2:, :],
                            (((1,), (0,)), ((), ())),
                            preferred_element_type=jnp.float32)
          + b0_ref[...])
    y0_ref[0] = y0

    @pl.when(jnp.logical_and(bi == 0, ti == 0))
    def _init():
        s_ref[...] = jnp.zeros_like(s_ref)
        q_ref[...] = jnp.zeros_like(q_ref)

    s_ref[...] += jnp.sum(y0, axis=0, keepdims=True)
    q_ref[...] += jnp.sum(y0 * y0, axis=0, keepdims=True)


def _mlp1_body(y0_ref, s0_ref, q0_ref, g0_ref, be0_ref, w1t_ref, b1_ref,
               y1_ref, s_ref, q_ref, *, inv_cnt):
    bi = pl.program_id(0)
    ti = pl.program_id(1)
    mean0 = s0_ref[...] * inv_cnt
    var0 = q0_ref[...] * inv_cnt - mean0 * mean0
    a0 = g0_ref[...] / jnp.sqrt(var0 + 1e-5)
    c0 = be0_ref[...] - mean0 * a0
    h = jnp.maximum(y0_ref[0] * a0 + c0, 0.0)
    y1 = lax.dot_general(h, w1t_ref[...], (((1,), (0,)), ((), ())),
                         preferred_element_type=jnp.float32) + b1_ref[...]
    y1_ref[0] = y1

    @pl.when(jnp.logical_and(bi == 0, ti == 0))
    def _init():
        s_ref[...] = jnp.zeros_like(s_ref)
        q_ref[...] = jnp.zeros_like(q_ref)

    s_ref[...] += jnp.sum(y1, axis=0, keepdims=True)
    q_ref[...] += jnp.sum(y1 * y1, axis=0, keepdims=True)


def _finalize_body(y1_ref, s1_ref, q1_ref, g1_ref, be1_ref, out_ref, *,
                   inv_cnt):
    mean1 = s1_ref[...] * inv_cnt
    var1 = q1_ref[...] * inv_cnt - mean1 * mean1
    a1 = g1_ref[...] / jnp.sqrt(var1 + 1e-5)
    c1 = be1_ref[...] - mean1 * a1
    z = jnp.maximum(y1_ref[0] * a1 + c1, 0.0)
    out_ref[0] = z.T


@jax.jit
def _run(xyz1, xyz2, points1, points2, W0, b0, g0, be0, W1, b1, g1, be1):
    b, n, _ = xyz1.shape
    m = xyz2.shape[1]
    c1 = points1.shape[2]
    c2 = points2.shape[2]
    cin = c1 + c2
    co0 = W0.shape[0]
    co1 = W1.shape[0]
    f32 = jnp.float32

    tn = min(2048, n)
    grid = (b, n // tn)

    xyz2t = jnp.transpose(xyz2, (0, 2, 1))
    w0t = W0.T
    w1t = W1.T
    b0r = b0.reshape(1, co0)
    b1r = b1.reshape(1, co1)

    idx3, wts3 = pl.pallas_call(
        functools.partial(_knn_body, m=m),
        grid=grid,
        in_specs=[
            pl.BlockSpec((1, tn, 3), lambda bi, ti: (bi, ti, 0)),
            pl.BlockSpec((1, 3, m), lambda bi, ti: (bi, 0, 0)),
        ],
        out_specs=[
            pl.BlockSpec((1, 3, tn), lambda bi, ti: (bi, 0, ti)),
            pl.BlockSpec((1, tn, 3), lambda bi, ti: (bi, ti, 0)),
        ],
        out_shape=[
            jax.ShapeDtypeStruct((b, 3, n), jnp.int32),
            jax.ShapeDtypeStruct((b, n, 3), f32),
        ],
        compiler_params=pltpu.CompilerParams(
            vmem_limit_bytes=112 * 1024 * 1024),
    )(xyz1, xyz2t)

    idx_flat = idx3.reshape(3 * b * n)
    table = points2.reshape(b * m, c2)
    gathered = _make_sc_gather(3 * b * n, c2, 32)(idx_flat, table)
    g4 = gathered.reshape(b, 3, n, c2)

    y0, s0, q0 = pl.pallas_call(
        functools.partial(_interp_mlp0_body, c2=c2),
        grid=grid,
        in_specs=[
            pl.BlockSpec((1, 3, tn, c2), lambda bi, ti: (bi, 0, ti, 0)),
            pl.BlockSpec((1, tn, 3), lambda bi, ti: (bi, ti, 0)),
            pl.BlockSpec((1, tn, c1), lambda bi, ti: (bi, ti, 0)),
            pl.BlockSpec((cin, co0), lambda bi, ti: (0, 0)),
            pl.BlockSpec((1, co0), lambda bi, ti: (0, 0)),
        ],
        out_specs=[
            pl.BlockSpec((1, tn, co0), lambda bi, ti: (bi, ti, 0)),
            pl.BlockSpec((1, co0), lambda bi, ti: (0, 0)),
            pl.BlockSpec((1, co0), lambda bi, ti: (0, 0)),
        ],
        out_shape=[
            jax.ShapeDtypeStruct((b, n, co0), f32),
            jax.ShapeDtypeStruct((1, co0), f32),
            jax.ShapeDtypeStruct((1, co0), f32),
        ],
    )(g4, wts3, points1, w0t, b0r)

    inv_cnt = 1.0 / float(b * n)
    g0r = g0.reshape(1, co0)
    be0r = be0.reshape(1, co0)
    g1r = g1.reshape(1, co1)
    be1r = be1.reshape(1, co1)

    y1, s1, q1 = pl.pallas_call(
        functools.partial(_mlp1_body, inv_cnt=inv_cnt),
        grid=grid,
        in_specs=[
            pl.BlockSpec((1, tn, co0), lambda bi, ti: (bi, ti, 0)),
            pl.BlockSpec((1, co0), lambda bi, ti: (0, 0)),
            pl.BlockSpec((1, co0), lambda bi, ti: (0, 0)),
            pl.BlockSpec((1, co0), lambda bi, ti: (0, 0)),
            pl.BlockSpec((1, co0), lambda bi, ti: (0, 0)),
            pl.BlockSpec((co0, co1), lambda bi, ti: (0, 0)),
            pl.BlockSpec((1, co1), lambda bi, ti: (0, 0)),
        ],
        out_specs=[
            pl.BlockSpec((1, tn, co1), lambda bi, ti: (bi, ti, 0)),
            pl.BlockSpec((1, co1), lambda bi, ti: (0, 0)),
            pl.BlockSpec((1, co1), lambda bi, ti: (0, 0)),
        ],
        out_shape=[
            jax.ShapeDtypeStruct((b, n, co1), f32),
            jax.ShapeDtypeStruct((1, co1), f32),
            jax.ShapeDtypeStruct((1, co1), f32),
        ],
    )(y0, s0, q0, g0r, be0r, w1t, b1r)

    out = pl.pallas_call(
        functools.partial(_finalize_body, inv_cnt=inv_cnt),
        grid=grid,
        in_specs=[
            pl.BlockSpec((1, tn, co1), lambda bi, ti: (bi, ti, 0)),
            pl.BlockSpec((1, co1), lambda bi, ti: (0, 0)),
            pl.BlockSpec((1, co1), lambda bi, ti: (0, 0)),
            pl.BlockSpec((1, co1), lambda bi, ti: (0, 0)),
            pl.BlockSpec((1, co1), lambda bi, ti: (0, 0)),
        ],
        out_specs=pl.BlockSpec((1, co1, tn), lambda bi, ti: (bi, 0, ti)),
        out_shape=jax.ShapeDtypeStruct((b, co1, n), f32),
    )(y1, s1, q1, g1r, be1r)

    return out


def kernel(xyz1, xyz2, points1, points2, W0, b0, g0, be0, W1, b1, g1, be1):
    return _run(xyz1, xyz2, points1, points2, W0, b0, g0, be0,
                W1, b1, g1, be1)
